# baseline (speedup 1.0000x reference)
import functools

import jax
import jax.numpy as jnp
from jax import lax
from jax.experimental import pallas as pl
from jax.experimental.pallas import tpu as pltpu
from jax.experimental.pallas import tpu_sc as plsc
from jax.experimental import layout as jax_layout


def _build(B, L, D):
    info = plsc.get_sparse_core_info()
    NC, NS = info.num_cores, info.num_subcores
    NW = NC * NS
    BW = B // NW
    CB = 2
    NCHUNK = BW // CB
    G1 = 104
    G2 = L - G1

    mesh = plsc.VectorSubcoreMesh(core_axis_name="c", subcore_axis_name="s")

    @functools.partial(
        pl.kernel,
        out_type=jax.ShapeDtypeStruct((B, L, 2 * D), jnp.float32),
        mesh=mesh,
        scratch_types=[
            pltpu.VMEM((BW, L), jnp.int32),
            pltpu.VMEM((BW, L), jnp.int32),
            pltpu.VMEM((2, CB, L, D), jnp.float32),
            pltpu.VMEM((2, CB, L, D), jnp.float32),
            pltpu.SemaphoreType.DMA((2,)),
            pltpu.SemaphoreType.DMA((2,)),
        ],
        compiler_params=pltpu.CompilerParams(use_tc_tiling_on_sc=False),
    )
    def emb(pos_hbm, ner_hbm, tpos_hbm, tner_hbm, out_hbm,
            idx_p, idx_n, rows_p, rows_n, gsem, wsem):
        wid = lax.axis_index("s") * NC + lax.axis_index("c")
        b0 = wid * BW

        pltpu.sync_copy(pos_hbm.at[pl.ds(b0, BW)], idx_p)
        pltpu.sync_copy(ner_hbm.at[pl.ds(b0, BW)], idx_n)

        def fire_gathers(c, b):
            for rl in range(CB):
                r = c * CB + rl
                for off, g in ((0, G1), (G1, G2)):
                    pltpu.async_copy(tpos_hbm.at[idx_p.at[r, pl.ds(off, g)]],
                                     rows_p.at[b, rl, pl.ds(off, g)], gsem.at[b])
                    pltpu.async_copy(tner_hbm.at[idx_n.at[r, pl.ds(off, g)]],
                                     rows_n.at[b, rl, pl.ds(off, g)], gsem.at[b])

        def drain_gathers(b):
            for rl in range(CB):
                for off, g in ((0, G1), (G1, G2)):
                    pltpu.make_async_copy(
                        tpos_hbm.at[idx_p.at[0, pl.ds(off, g)]],
                        rows_p.at[b, 0, pl.ds(off, g)], gsem.at[b]).wait()
                    pltpu.make_async_copy(
                        tner_hbm.at[idx_n.at[0, pl.ds(off, g)]],
                        rows_n.at[b, 0, pl.ds(off, g)], gsem.at[b]).wait()

        def fire_writes(c, b):
            bg = b0 + c * CB
            pltpu.async_copy(rows_p.at[b],
                             out_hbm.at[pl.ds(bg, CB), :, pl.ds(0, D)],
                             wsem.at[b])
            pltpu.async_copy(rows_n.at[b],
                             out_hbm.at[pl.ds(bg, CB), :, pl.ds(D, D)],
                             wsem.at[b])

        def drain_writes(b):
            pltpu.make_async_copy(rows_p.at[b],
                                  out_hbm.at[pl.ds(0, CB), :, pl.ds(0, D)],
                                  wsem.at[b]).wait()
            pltpu.make_async_copy(rows_n.at[b],
                                  out_hbm.at[pl.ds(0, CB), :, pl.ds(D, D)],
                                  wsem.at[b]).wait()

        fire_gathers(0, 0)
        fire_gathers(1, 1)

        def pair(cc, carry):
            for b in range(2):
                c = 2 * cc + b
                drain_gathers(b)
                fire_writes(c, b)
                drain_writes(b)
                @pl.when(c + 2 < NCHUNK)
                def _():
                    fire_gathers(c + 2, b)
            return carry

        pl.loop(0, NCHUNK // 2)(lambda cc: pair(cc, None))

    return emb


@jax.jit
def kernel(pos_ids, ner_ids, table_pos, table_ner):
    B, L = pos_ids.shape
    V, D = table_pos.shape
    out = _build(B, L, D)(pos_ids, ner_ids, table_pos, table_ner)
    return jax_layout.with_layout_constraint(
        out, jax_layout.Layout(major_to_minor=(0, 1, 2), tiling=()))
